# Initial kernel scaffold; baseline (speedup 1.0000x reference)
import jax
import jax.numpy as jnp
from jax.experimental import pallas as pl


def kernel(xs, table):
    raise NotImplementedError("write your pallas kernel here")


# speedup vs baseline: 1.4885x; 1.4885x over previous
import functools

import jax
import jax.numpy as jnp
from jax import lax
from jax.experimental import pallas as pl
from jax.experimental.pallas import tpu as pltpu
from jax.experimental.pallas import tpu_sc as plsc

D = 32
G = 128
K = 20
NW = 32
B_TOTAL = 4096 * 200
B_PER_W = B_TOTAL // NW
C = K * G
NCHUNK = B_PER_W // C

_mesh = plsc.VectorSubcoreMesh(core_axis_name="c", subcore_axis_name="s")


@functools.partial(
    pl.kernel,
    out_type=jax.ShapeDtypeStruct((B_TOTAL, D), jnp.float32),
    mesh=_mesh,
    scratch_types=[
        pltpu.VMEM((C,), jnp.int32),
        pltpu.VMEM((C, D), jnp.float32),
        pltpu.SemaphoreType.DMA,
    ],
    compiler_params=pltpu.CompilerParams(use_tc_tiling_on_sc=False),
)
def _emb_lookup(xs_hbm, table_hbm, out_hbm, idx_v, rows_v, sem):
    wid = lax.axis_index("s") * 2 + lax.axis_index("c")
    base = wid * B_PER_W

    @pl.loop(0, NCHUNK)
    def _chunk(ci):
        off = base + ci * C
        pltpu.sync_copy(xs_hbm.at[pl.ds(off, C)], idx_v)
        copies = []
        for j in range(K):
            copies.append(
                pltpu.async_copy(
                    table_hbm.at[idx_v.at[pl.ds(j * G, G)]],
                    rows_v.at[pl.ds(j * G, G)],
                    sem,
                )
            )
        for c in copies:
            c.wait()
        pltpu.sync_copy(rows_v, out_hbm.at[pl.ds(off, C)])


def kernel(xs, table):
    out = _emb_lookup(xs.reshape(B_TOTAL), table)
    return out.reshape(4096, 200, D)


# speedup vs baseline: 1.4992x; 1.0072x over previous
import functools

import jax
import jax.numpy as jnp
from jax import lax
from jax.experimental import pallas as pl
from jax.experimental.pallas import tpu as pltpu
from jax.experimental.pallas import tpu_sc as plsc

D = 32
G = 128
K = 10
NW = 32
B_TOTAL = 4096 * 200
B_PER_W = B_TOTAL // NW
C = K * G
NCHUNK = B_PER_W // C

_mesh = plsc.VectorSubcoreMesh(core_axis_name="c", subcore_axis_name="s")


@functools.partial(
    pl.kernel,
    out_type=jax.ShapeDtypeStruct((B_TOTAL, D), jnp.float32),
    mesh=_mesh,
    scratch_types=[
        pltpu.VMEM((B_PER_W,), jnp.int32),
        pltpu.VMEM((2, C, D), jnp.float32),
        pltpu.SemaphoreType.DMA,
        pltpu.SemaphoreType.DMA,
        pltpu.SemaphoreType.DMA,
        pltpu.SemaphoreType.DMA,
    ],
    compiler_params=pltpu.CompilerParams(use_tc_tiling_on_sc=False),
)
def _emb_lookup(xs_hbm, table_hbm, out_hbm, idx_v, rows_v, g0, g1, w0, w1):
    wid = lax.axis_index("s") * 2 + lax.axis_index("c")
    base = wid * B_PER_W
    gsem = (g0, g1)
    wsem = (w0, w1)

    pltpu.sync_copy(xs_hbm.at[pl.ds(base, B_PER_W)], idx_v)

    def fire_gathers(ci, b):
        for j in range(K):
            pltpu.async_copy(
                table_hbm.at[idx_v.at[pl.ds(ci * C + j * G, G)]],
                rows_v.at[b].at[pl.ds(j * G, G)],
                gsem[b],
            )

    def wait_gathers(b):
        for j in range(K):
            pltpu.make_async_copy(
                table_hbm.at[idx_v.at[pl.ds(j * G, G)]],
                rows_v.at[b].at[pl.ds(j * G, G)],
                gsem[b],
            ).wait()

    def fire_write(ci, b):
        pltpu.async_copy(
            rows_v.at[b], out_hbm.at[pl.ds(base + ci * C, C)], wsem[b]
        )

    def wait_write(b):
        pltpu.make_async_copy(
            rows_v.at[b], out_hbm.at[pl.ds(base, C)], wsem[b]
        ).wait()

    fire_gathers(0, 0)

    @pl.loop(0, NCHUNK, step=2)
    def _step(ci):
        @pl.when(ci > 0)
        def _():
            wait_write(1)
        fire_gathers(ci + 1, 1)
        wait_gathers(0)
        fire_write(ci, 0)
        @pl.when(ci + 2 < NCHUNK)
        def _():
            wait_write(0)
            fire_gathers(ci + 2, 0)
        wait_gathers(1)
        fire_write(ci + 1, 1)

    wait_write(0)
    wait_write(1)


def kernel(xs, table):
    out = _emb_lookup(xs.reshape(B_TOTAL), table)
    return out.reshape(4096, 200, D)
